# Initial kernel scaffold; baseline (speedup 1.0000x reference)
import jax
import jax.numpy as jnp
from jax.experimental import pallas as pl


def kernel(x, edge_index, WQ, bQ, WK, bK, WV, bV, Wout, bout, ln1_w, ln1_b, ln2_w, ln2_b):
    raise NotImplementedError("write your pallas kernel here")


# speedup vs baseline: 5.2114x; 5.2114x over previous
import functools

import numpy as np
import jax
import jax.numpy as jnp
from jax import lax
from jax.experimental import pallas as pl
from jax.experimental.pallas import tpu as pltpu
from jax.experimental.pallas import tpu_sc as plsc

N = 10000
E = 320000
D = 128
H = 8
HD = D // H
EPS = 1e-5

_info = plsc.get_sparse_core_info()
NC = _info.num_cores
NS = _info.num_subcores
L = _info.num_lanes
NW = NC * NS
EPW = E // NW
C = 80
NCHUNK = EPW // C
NPT = N // NS

_PERM = np.empty((D,), np.int64)
for _s in range(8):
    for _l in range(16):
        _PERM[16 * _s + _l] = (_l % 8) * 16 + 2 * _s + _l // 8

_S128 = np.zeros((D, D), np.float32)
for _j in range(D):
    for _p in range(D):
        if _j % 8 == _p % 8:
            _S128[_j, _p] = 0.0625

_mesh = plsc.VectorSubcoreMesh(core_axis_name="c", subcore_axis_name="s")


@functools.partial(
    pl.kernel,
    mesh=_mesh,
    out_type=[
        jax.ShapeDtypeStruct((E, L), jnp.float32),
        jax.ShapeDtypeStruct((NW, L), jnp.float32),
    ],
    scratch_types=[
        pltpu.VMEM((C,), jnp.int32),
        pltpu.VMEM((C,), jnp.int32),
        pltpu.VMEM((C, D), jnp.float32),
        pltpu.VMEM((C, D), jnp.float32),
        pltpu.VMEM((C, L), jnp.float32),
        pltpu.VMEM((2 * L,), jnp.float32),
        pltpu.VMEM((L,), jnp.float32),
        pltpu.SemaphoreType.DMA,
        pltpu.SemaphoreType.DMA,
    ],
)
def _logits_pass(src_hbm, dst_hbm, qx_hbm, kx_hbm, lg_hbm, mx_hbm,
                 sidx, didx, qbuf, kbuf, lbuf, rot, rbuf, sem1, sem2):
    wid = lax.axis_index("s") * NC + lax.axis_index("c")

    def chunk(i, rmax):
        base = wid * EPW + i * C
        pltpu.sync_copy(src_hbm.at[pl.ds(base, C)], sidx)
        pltpu.sync_copy(dst_hbm.at[pl.ds(base, C)], didx)
        cq = pltpu.async_copy(qx_hbm.at[sidx], qbuf, sem1)
        ck = pltpu.async_copy(kx_hbm.at[didx], kbuf, sem2)
        cq.wait()
        ck.wait()

        def edge(e, rm):
            w = qbuf[e, pl.ds(0, L)] * kbuf[e, pl.ds(0, L)]
            for s in range(1, 8):
                w = w + qbuf[e, pl.ds(L * s, L)] * kbuf[e, pl.ds(L * s, L)]
            rot[pl.ds(0, L)] = w
            rot[pl.ds(L, L)] = w
            lv = (w + rot[pl.ds(8, L)]) * 0.25
            lbuf[e, :] = lv
            return jnp.maximum(rm, lv)

        rmax = lax.fori_loop(0, C, edge, rmax)
        pltpu.sync_copy(lbuf, lg_hbm.at[pl.ds(base, C)])
        return rmax

    rmax = lax.fori_loop(0, NCHUNK, chunk,
                         jnp.full((L,), -jnp.inf, jnp.float32))
    rbuf[...] = rmax
    pltpu.sync_copy(rbuf, mx_hbm.at[wid])


@functools.partial(
    pl.kernel,
    mesh=_mesh,
    out_type=jax.ShapeDtypeStruct((NC * N, D), jnp.float32),
    scratch_types=[
        pltpu.VMEM((C,), jnp.int32),
        pltpu.VMEM((C,), jnp.int32),
        pltpu.VMEM((C, D), jnp.float32),
        pltpu.VMEM((C, D), jnp.float32),
        pltpu.VMEM((C, L), jnp.float32),
        pltpu.VMEM((NW, L), jnp.float32),
        pltpu.VMEM((2 * L,), jnp.float32),
        pltpu.VMEM_SHARED((N, D), jnp.float32),
        pltpu.SemaphoreType.DMA,
        pltpu.SemaphoreType.DMA,
    ],
)
def _aggregate_pass(src_hbm, dst_hbm, vx_hbm, lg_hbm, mx_hbm, znd_hbm,
                    acc_hbm,
                    sidx, didx, vbuf, wbuf, lbuf, mbuf, rot,
                    acc_sh, sem1, sem2):
    cid = lax.axis_index("c")
    sid = lax.axis_index("s")
    wid = sid * NC + cid

    pltpu.sync_copy(mx_hbm, mbuf)
    gv = mbuf[0, :]
    for i in range(1, NW):
        gv = jnp.maximum(gv, mbuf[i, :])
    for off in (8, 4, 2, 1):
        rot[pl.ds(0, L)] = gv
        rot[pl.ds(L, L)] = gv
        gv = jnp.maximum(gv, rot[pl.ds(off, L)])

    row0 = sid * 624
    pltpu.sync_copy(znd_hbm.at[pl.ds(row0, 640)], acc_sh.at[pl.ds(row0, 640)])
    plsc.subcore_barrier()

    def chunk(i, carry):
        base = wid * EPW + i * C
        pltpu.sync_copy(dst_hbm.at[pl.ds(base, C)], didx)
        cv = pltpu.async_copy(vx_hbm.at[didx], vbuf, sem1)
        pltpu.sync_copy(src_hbm.at[pl.ds(base, C)], sidx)
        pltpu.sync_copy(lg_hbm.at[pl.ds(base, C)], lbuf)
        cv.wait()

        def edge(e, c2):
            ex = jnp.exp(lbuf[e, :] - gv)
            for s in range(8):
                wbuf[e, pl.ds(L * s, L)] = vbuf[e, pl.ds(L * s, L)] * ex
            return c2

        lax.fori_loop(0, C, edge, 0)
        pltpu.sync_copy(wbuf, acc_sh.at[sidx], add=True)
        return carry

    lax.fori_loop(0, NCHUNK, chunk, 0)
    plsc.subcore_barrier()

    @pl.when(sid == 0)
    def _():
        pltpu.sync_copy(acc_sh, acc_hbm.at[pl.ds(cid * N, N)])


@functools.partial(
    pl.kernel,
    mesh=_mesh,
    out_type=jax.ShapeDtypeStruct((NC * N, D), jnp.float32),
    scratch_types=[
        pltpu.VMEM((C,), jnp.int32),
        pltpu.VMEM((C, L), jnp.float32),
        pltpu.VMEM((C, D), jnp.float32),
        pltpu.VMEM((NW, L), jnp.float32),
        pltpu.VMEM((2 * L,), jnp.float32),
        pltpu.VMEM_SHARED((N, D), jnp.float32),
    ],
)
def _denom_pass(src_hbm, lg_hbm, mx_hbm, znl_hbm, den_hbm,
                sidx, lbuf, exbuf, mbuf, rot, den_sh):
    cid = lax.axis_index("c")
    sid = lax.axis_index("s")
    wid = sid * NC + cid

    pltpu.sync_copy(mx_hbm, mbuf)
    gv = mbuf[0, :]
    for i in range(1, NW):
        gv = jnp.maximum(gv, mbuf[i, :])
    for off in (8, 4, 2, 1):
        rot[pl.ds(0, L)] = gv
        rot[pl.ds(L, L)] = gv
        gv = jnp.maximum(gv, rot[pl.ds(off, L)])

    row0 = sid * 624
    pltpu.sync_copy(znl_hbm.at[pl.ds(row0, 640)], den_sh.at[pl.ds(row0, 640)])
    plsc.subcore_barrier()

    def chunk(i, carry):
        base = wid * EPW + i * C
        pltpu.sync_copy(src_hbm.at[pl.ds(base, C)], sidx)
        pltpu.sync_copy(lg_hbm.at[pl.ds(base, C)], lbuf)

        def edge(e, c2):
            ex = jnp.exp(lbuf[e, :] - gv)
            for s in range(8):
                exbuf[e, pl.ds(L * s, L)] = ex
            return c2

        lax.fori_loop(0, C, edge, 0)
        pltpu.sync_copy(exbuf, den_sh.at[sidx], add=True)
        return carry

    lax.fori_loop(0, NCHUNK, chunk, 0)
    plsc.subcore_barrier()

    @pl.when(sid == 0)
    def _():
        pltpu.sync_copy(den_sh, den_hbm.at[pl.ds(cid * N, N)])


RB = 2000


def _ln_rows(v, w, b):
    mu = jnp.mean(v, axis=-1, keepdims=True)
    var = jnp.mean((v - mu) ** 2, axis=-1, keepdims=True)
    return (v - mu) / jnp.sqrt(var + EPS) * w + b


def _proj_body(x_ref, wq, wk, wv, bq, bk, bv, q_out, k_out, v_out):
    xb = x_ref[...]
    q_out[...] = jnp.dot(xb, wq[...], preferred_element_type=jnp.float32) + bq[...]
    k_out[...] = jnp.dot(xb, wk[...], preferred_element_type=jnp.float32) + bk[...]
    v_out[...] = jnp.dot(xb, wv[...], preferred_element_type=jnp.float32) + bv[...]


def _final_body(acc_ref, den_ref, s_ref, wout, bout, l1w, l1b, l2w, l2b, o_ref):
    acc = acc_ref[0] + acc_ref[1]
    den = den_ref[0] + den_ref[1]
    div = jnp.dot(den, s_ref[...], preferred_element_type=jnp.float32)
    attn = acc / (div + 1e-16)
    h1 = _ln_rows(attn, l1w[...], l1b[...])
    h2 = jnp.dot(h1, wout[...], preferred_element_type=jnp.float32) + bout[...]
    o_ref[...] = _ln_rows(h2, l2w[...], l2b[...])


def kernel(x, edge_index, WQ, bQ, WK, bK, WV, bV, Wout, bout,
           ln1_w, ln1_b, ln2_w, ln2_b):
    src = edge_index[0].astype(jnp.int32)
    dst = edge_index[1].astype(jnp.int32)

    def b2(v):
        return v.reshape(1, D).astype(jnp.float32)

    qx, kx, vx = pl.pallas_call(
        _proj_body,
        grid=(N // RB,),
        in_specs=[pl.BlockSpec((RB, D), lambda i: (i, 0))]
        + [pl.BlockSpec((D, D), lambda i: (0, 0))] * 3
        + [pl.BlockSpec((1, D), lambda i: (0, 0))] * 3,
        out_specs=[pl.BlockSpec((RB, D), lambda i: (i, 0))] * 3,
        out_shape=[jax.ShapeDtypeStruct((N, D), jnp.float32)] * 3,
    )(x, WQ[:, _PERM], WK[:, _PERM], WV[:, _PERM],
      b2(bQ[_PERM]), b2(bK[_PERM]), b2(bV[_PERM]))

    lg, mx = _logits_pass(src, dst, qx, kx)

    znd = jnp.zeros((N, D), jnp.float32)
    accp = _aggregate_pass(src, dst, vx, lg, mx, znd)
    denp = _denom_pass(src, lg, mx, znd)
    acc3 = accp.reshape(NC, N, D)
    den3 = denp.reshape(NC, N, D)

    out = pl.pallas_call(
        _final_body,
        grid=(N // RB,),
        in_specs=[
            pl.BlockSpec((NC, RB, D), lambda i: (0, i, 0)),
            pl.BlockSpec((NC, RB, D), lambda i: (0, i, 0)),
            pl.BlockSpec((D, D), lambda i: (0, 0)),
            pl.BlockSpec((D, D), lambda i: (0, 0)),
            pl.BlockSpec((1, D), lambda i: (0, 0)),
            pl.BlockSpec((1, D), lambda i: (0, 0)),
            pl.BlockSpec((1, D), lambda i: (0, 0)),
            pl.BlockSpec((1, D), lambda i: (0, 0)),
            pl.BlockSpec((1, D), lambda i: (0, 0)),
        ],
        out_specs=pl.BlockSpec((RB, D), lambda i: (i, 0)),
        out_shape=jax.ShapeDtypeStruct((N, D), jnp.float32),
    )(acc3, den3, _S128, Wout[_PERM, :], b2(bout),
      b2(ln1_w[_PERM]), b2(ln1_b[_PERM]), b2(ln2_w), b2(ln2_b))
    return out


# speedup vs baseline: 6.2676x; 1.2027x over previous
import functools

import numpy as np
import jax
import jax.numpy as jnp
from jax import lax
from jax.experimental import pallas as pl
from jax.experimental.pallas import tpu as pltpu
from jax.experimental.pallas import tpu_sc as plsc

N = 10000
E = 320000
D = 128
H = 8
HD = D // H
EPS = 1e-5

_info = plsc.get_sparse_core_info()
NC = _info.num_cores
NS = _info.num_subcores
L = _info.num_lanes
NW = NC * NS
EPW = E // NW
C = 80
NCHUNK = EPW // C
NPT = N // NS

_PERM = np.empty((D,), np.int64)
for _s in range(8):
    for _l in range(16):
        _PERM[16 * _s + _l] = (_l % 8) * 16 + 2 * _s + _l // 8

_S128 = np.zeros((D, D), np.float32)
for _j in range(D):
    for _p in range(D):
        if _j % 8 == _p % 8:
            _S128[_j, _p] = 0.0625

_mesh = plsc.VectorSubcoreMesh(core_axis_name="c", subcore_axis_name="s")


@functools.partial(
    pl.kernel,
    mesh=_mesh,
    out_type=[
        jax.ShapeDtypeStruct((E, L), jnp.float32),
        jax.ShapeDtypeStruct((NW, L), jnp.float32),
    ],
    scratch_types=[
        pltpu.VMEM((C,), jnp.int32),
        pltpu.VMEM((C,), jnp.int32),
        pltpu.VMEM((C,), jnp.int32),
        pltpu.VMEM((C,), jnp.int32),
        pltpu.VMEM((C, D), jnp.float32),
        pltpu.VMEM((C, D), jnp.float32),
        pltpu.VMEM((C, D), jnp.float32),
        pltpu.VMEM((C, D), jnp.float32),
        pltpu.VMEM((C, L), jnp.float32),
        pltpu.VMEM((C, L), jnp.float32),
        pltpu.VMEM((2 * L,), jnp.float32),
        pltpu.VMEM((L,), jnp.float32),
        pltpu.SemaphoreType.DMA,
        pltpu.SemaphoreType.DMA,
        pltpu.SemaphoreType.DMA,
        pltpu.SemaphoreType.DMA,
        pltpu.SemaphoreType.DMA,
        pltpu.SemaphoreType.DMA,
    ],
)
def _logits_pass(src_hbm, dst_hbm, qx_hbm, kx_hbm, lg_hbm, mx_hbm,
                 si0, si1, di0, di1, qb0, qb1, kb0, kb1, lb0, lb1,
                 rot, rbuf, sq0, sq1, sk0, sk1, sl0, sl1):
    wid = lax.axis_index("s") * NC + lax.axis_index("c")
    sets = ((si0, di0, qb0, kb0, lb0, sq0, sk0, sl0),
            (si1, di1, qb1, kb1, lb1, sq1, sk1, sl1))

    def start(ci, S):
        si, di, qb, kb, lb, sq, sk, sl = S
        base = wid * EPW + ci * C
        pltpu.sync_copy(src_hbm.at[pl.ds(base, C)], si)
        pltpu.sync_copy(dst_hbm.at[pl.ds(base, C)], di)
        pltpu.async_copy(qx_hbm.at[si], qb, sq)
        pltpu.async_copy(kx_hbm.at[di], kb, sk)

    def step(ci, S, rm):
        si, di, qb, kb, lb, sq, sk, sl = S
        pltpu.make_async_copy(qx_hbm.at[si], qb, sq).wait()
        pltpu.make_async_copy(kx_hbm.at[di], kb, sk).wait()

        @pl.when(ci >= 2)
        def _():
            base_p = wid * EPW + (ci - 2) * C
            pltpu.make_async_copy(lb, lg_hbm.at[pl.ds(base_p, C)], sl).wait()

        def edge(e, rm):
            w = qb[e, pl.ds(0, L)] * kb[e, pl.ds(0, L)]
            for s in range(1, 8):
                w = w + qb[e, pl.ds(L * s, L)] * kb[e, pl.ds(L * s, L)]
            rot[pl.ds(0, L)] = w
            rot[pl.ds(L, L)] = w
            lv = (w + rot[pl.ds(8, L)]) * 0.25
            lb[e, :] = lv
            return jnp.maximum(rm, lv)

        rm = lax.fori_loop(0, C, edge, rm)
        base = wid * EPW + ci * C
        pltpu.async_copy(lb, lg_hbm.at[pl.ds(base, C)], sl)

        @pl.when(ci + 2 < NCHUNK)
        def _():
            start(ci + 2, S)

        return rm

    start(0, sets[0])
    start(1, sets[1])

    def pair(k, rm):
        rm = step(2 * k, sets[0], rm)
        rm = step(2 * k + 1, sets[1], rm)
        return rm

    rmax = lax.fori_loop(0, NCHUNK // 2, pair,
                         jnp.full((L,), -jnp.inf, jnp.float32))
    rmax = step(NCHUNK - 1, sets[0], rmax)
    b1 = wid * EPW + (NCHUNK - 2) * C
    pltpu.make_async_copy(lb1, lg_hbm.at[pl.ds(b1, C)], sl1).wait()
    b0 = wid * EPW + (NCHUNK - 1) * C
    pltpu.make_async_copy(lb0, lg_hbm.at[pl.ds(b0, C)], sl0).wait()
    rbuf[...] = rmax
    pltpu.sync_copy(rbuf, mx_hbm.at[wid])


@functools.partial(
    pl.kernel,
    mesh=_mesh,
    out_type=jax.ShapeDtypeStruct((NC * N, D), jnp.float32),
    scratch_types=[
        pltpu.VMEM((C,), jnp.int32),
        pltpu.VMEM((C,), jnp.int32),
        pltpu.VMEM((C, D), jnp.float32),
        pltpu.VMEM((C, D), jnp.float32),
        pltpu.VMEM((C, L), jnp.float32),
        pltpu.VMEM((NW, L), jnp.float32),
        pltpu.VMEM((2 * L,), jnp.float32),
        pltpu.VMEM_SHARED((N, D), jnp.float32),
        pltpu.SemaphoreType.DMA,
        pltpu.SemaphoreType.DMA,
    ],
)
def _aggregate_pass(src_hbm, dst_hbm, vx_hbm, lg_hbm, mx_hbm, znd_hbm,
                    acc_hbm,
                    sidx, didx, vbuf, wbuf, lbuf, mbuf, rot,
                    acc_sh, sem1, sem2):
    cid = lax.axis_index("c")
    sid = lax.axis_index("s")
    wid = sid * NC + cid

    pltpu.sync_copy(mx_hbm, mbuf)
    gv = mbuf[0, :]
    for i in range(1, NW):
        gv = jnp.maximum(gv, mbuf[i, :])
    for off in (8, 4, 2, 1):
        rot[pl.ds(0, L)] = gv
        rot[pl.ds(L, L)] = gv
        gv = jnp.maximum(gv, rot[pl.ds(off, L)])

    row0 = sid * 624
    pltpu.sync_copy(znd_hbm.at[pl.ds(row0, 640)], acc_sh.at[pl.ds(row0, 640)])
    plsc.subcore_barrier()

    def chunk(i, carry):
        base = wid * EPW + i * C
        pltpu.sync_copy(dst_hbm.at[pl.ds(base, C)], didx)
        cv = pltpu.async_copy(vx_hbm.at[didx], vbuf, sem1)
        pltpu.sync_copy(src_hbm.at[pl.ds(base, C)], sidx)
        pltpu.sync_copy(lg_hbm.at[pl.ds(base, C)], lbuf)
        cv.wait()

        def edge(e, c2):
            ex = jnp.exp(lbuf[e, :] - gv)
            for s in range(8):
                wbuf[e, pl.ds(L * s, L)] = vbuf[e, pl.ds(L * s, L)] * ex
            return c2

        lax.fori_loop(0, C, edge, 0)
        pltpu.sync_copy(wbuf, acc_sh.at[sidx], add=True)
        return carry

    lax.fori_loop(0, NCHUNK, chunk, 0)
    plsc.subcore_barrier()

    @pl.when(sid == 0)
    def _():
        pltpu.sync_copy(acc_sh, acc_hbm.at[pl.ds(cid * N, N)])


@functools.partial(
    pl.kernel,
    mesh=_mesh,
    out_type=jax.ShapeDtypeStruct((NC * N, D), jnp.float32),
    scratch_types=[
        pltpu.VMEM((C,), jnp.int32),
        pltpu.VMEM((C, L), jnp.float32),
        pltpu.VMEM((C, D), jnp.float32),
        pltpu.VMEM((NW, L), jnp.float32),
        pltpu.VMEM((2 * L,), jnp.float32),
        pltpu.VMEM_SHARED((N, D), jnp.float32),
    ],
)
def _denom_pass(src_hbm, lg_hbm, mx_hbm, znl_hbm, den_hbm,
                sidx, lbuf, exbuf, mbuf, rot, den_sh):
    cid = lax.axis_index("c")
    sid = lax.axis_index("s")
    wid = sid * NC + cid

    pltpu.sync_copy(mx_hbm, mbuf)
    gv = mbuf[0, :]
    for i in range(1, NW):
        gv = jnp.maximum(gv, mbuf[i, :])
    for off in (8, 4, 2, 1):
        rot[pl.ds(0, L)] = gv
        rot[pl.ds(L, L)] = gv
        gv = jnp.maximum(gv, rot[pl.ds(off, L)])

    row0 = sid * 624
    pltpu.sync_copy(znl_hbm.at[pl.ds(row0, 640)], den_sh.at[pl.ds(row0, 640)])
    plsc.subcore_barrier()

    def chunk(i, carry):
        base = wid * EPW + i * C
        pltpu.sync_copy(src_hbm.at[pl.ds(base, C)], sidx)
        pltpu.sync_copy(lg_hbm.at[pl.ds(base, C)], lbuf)

        def edge(e, c2):
            ex = jnp.exp(lbuf[e, :] - gv)
            for s in range(8):
                exbuf[e, pl.ds(L * s, L)] = ex
            return c2

        lax.fori_loop(0, C, edge, 0)
        pltpu.sync_copy(exbuf, den_sh.at[sidx], add=True)
        return carry

    lax.fori_loop(0, NCHUNK, chunk, 0)
    plsc.subcore_barrier()

    @pl.when(sid == 0)
    def _():
        pltpu.sync_copy(den_sh, den_hbm.at[pl.ds(cid * N, N)])


RB = 2000


def _ln_rows(v, w, b):
    mu = jnp.mean(v, axis=-1, keepdims=True)
    var = jnp.mean((v - mu) ** 2, axis=-1, keepdims=True)
    return (v - mu) / jnp.sqrt(var + EPS) * w + b


def _proj_body(x_ref, wq, wk, wv, bq, bk, bv, q_out, k_out, v_out):
    xb = x_ref[...]
    q_out[...] = jnp.dot(xb, wq[...], preferred_element_type=jnp.float32) + bq[...]
    k_out[...] = jnp.dot(xb, wk[...], preferred_element_type=jnp.float32) + bk[...]
    v_out[...] = jnp.dot(xb, wv[...], preferred_element_type=jnp.float32) + bv[...]


def _final_body(acc_ref, den_ref, s_ref, wout, bout, l1w, l1b, l2w, l2b, o_ref):
    acc = acc_ref[0] + acc_ref[1]
    den = den_ref[0] + den_ref[1]
    div = jnp.dot(den, s_ref[...], preferred_element_type=jnp.float32)
    attn = acc / (div + 1e-16)
    h1 = _ln_rows(attn, l1w[...], l1b[...])
    h2 = jnp.dot(h1, wout[...], preferred_element_type=jnp.float32) + bout[...]
    o_ref[...] = _ln_rows(h2, l2w[...], l2b[...])


def kernel(x, edge_index, WQ, bQ, WK, bK, WV, bV, Wout, bout,
           ln1_w, ln1_b, ln2_w, ln2_b):
    src = edge_index[0].astype(jnp.int32)
    dst = edge_index[1].astype(jnp.int32)

    def b2(v):
        return v.reshape(1, D).astype(jnp.float32)

    qx, kx, vx = pl.pallas_call(
        _proj_body,
        grid=(N // RB,),
        in_specs=[pl.BlockSpec((RB, D), lambda i: (i, 0))]
        + [pl.BlockSpec((D, D), lambda i: (0, 0))] * 3
        + [pl.BlockSpec((1, D), lambda i: (0, 0))] * 3,
        out_specs=[pl.BlockSpec((RB, D), lambda i: (i, 0))] * 3,
        out_shape=[jax.ShapeDtypeStruct((N, D), jnp.float32)] * 3,
    )(x, WQ[:, _PERM], WK[:, _PERM], WV[:, _PERM],
      b2(bQ[_PERM]), b2(bK[_PERM]), b2(bV[_PERM]))

    lg, mx = _logits_pass(src, dst, qx, kx)

    znd = jnp.zeros((N, D), jnp.float32)
    accp = _aggregate_pass(src, dst, vx, lg, mx, znd)
    denp = _denom_pass(src, lg, mx, znd)
    acc3 = accp.reshape(NC, N, D)
    den3 = denp.reshape(NC, N, D)

    out = pl.pallas_call(
        _final_body,
        grid=(N // RB,),
        in_specs=[
            pl.BlockSpec((NC, RB, D), lambda i: (0, i, 0)),
            pl.BlockSpec((NC, RB, D), lambda i: (0, i, 0)),
            pl.BlockSpec((D, D), lambda i: (0, 0)),
            pl.BlockSpec((D, D), lambda i: (0, 0)),
            pl.BlockSpec((1, D), lambda i: (0, 0)),
            pl.BlockSpec((1, D), lambda i: (0, 0)),
            pl.BlockSpec((1, D), lambda i: (0, 0)),
            pl.BlockSpec((1, D), lambda i: (0, 0)),
            pl.BlockSpec((1, D), lambda i: (0, 0)),
        ],
        out_specs=pl.BlockSpec((RB, D), lambda i: (i, 0)),
        out_shape=jax.ShapeDtypeStruct((N, D), jnp.float32),
    )(acc3, den3, _S128, Wout[_PERM, :], b2(bout),
      b2(ln1_w[_PERM]), b2(ln1_b[_PERM]), b2(ln2_w), b2(ln2_b))
    return out


# speedup vs baseline: 6.8424x; 1.0917x over previous
import functools

import numpy as np
import jax
import jax.numpy as jnp
from jax import lax
from jax.experimental import pallas as pl
from jax.experimental.pallas import tpu as pltpu
from jax.experimental.pallas import tpu_sc as plsc

N = 10000
E = 320000
D = 128
H = 8
HD = D // H
EPS = 1e-5

_info = plsc.get_sparse_core_info()
NC = _info.num_cores
NS = _info.num_subcores
L = _info.num_lanes
NW = NC * NS
EPW = E // NW
C = 80
NCHUNK = EPW // C
NPT = N // NS

_PERM = np.empty((D,), np.int64)
for _s in range(8):
    for _l in range(16):
        _PERM[16 * _s + _l] = (_l % 8) * 16 + 2 * _s + _l // 8

_S128 = np.zeros((D, D), np.float32)
for _j in range(D):
    for _p in range(D):
        if _j % 8 == _p % 8:
            _S128[_j, _p] = 0.0625

_mesh = plsc.VectorSubcoreMesh(core_axis_name="c", subcore_axis_name="s")


@functools.partial(
    pl.kernel,
    mesh=_mesh,
    out_type=[
        jax.ShapeDtypeStruct((E, L), jnp.float32),
        jax.ShapeDtypeStruct((NW, L), jnp.float32),
    ],
    scratch_types=[
        pltpu.VMEM((C,), jnp.int32),
        pltpu.VMEM((C,), jnp.int32),
        pltpu.VMEM((C,), jnp.int32),
        pltpu.VMEM((C,), jnp.int32),
        pltpu.VMEM((C, D), jnp.float32),
        pltpu.VMEM((C, D), jnp.float32),
        pltpu.VMEM((C, D), jnp.float32),
        pltpu.VMEM((C, D), jnp.float32),
        pltpu.VMEM((C, L), jnp.float32),
        pltpu.VMEM((C, L), jnp.float32),
        pltpu.VMEM((2 * L,), jnp.float32),
        pltpu.VMEM((L,), jnp.float32),
        pltpu.SemaphoreType.DMA,
        pltpu.SemaphoreType.DMA,
        pltpu.SemaphoreType.DMA,
        pltpu.SemaphoreType.DMA,
        pltpu.SemaphoreType.DMA,
        pltpu.SemaphoreType.DMA,
    ],
)
def _logits_pass(src_hbm, dst_hbm, qx_hbm, kx_hbm, lg_hbm, mx_hbm,
                 si0, si1, di0, di1, qb0, qb1, kb0, kb1, lb0, lb1,
                 rot, rbuf, sq0, sq1, sk0, sk1, sl0, sl1):
    wid = lax.axis_index("s") * NC + lax.axis_index("c")
    sets = ((si0, di0, qb0, kb0, lb0, sq0, sk0, sl0),
            (si1, di1, qb1, kb1, lb1, sq1, sk1, sl1))

    def start(ci, S):
        si, di, qb, kb, lb, sq, sk, sl = S
        base = wid * EPW + ci * C
        pltpu.sync_copy(src_hbm.at[pl.ds(base, C)], si)
        pltpu.sync_copy(dst_hbm.at[pl.ds(base, C)], di)
        pltpu.async_copy(qx_hbm.at[si], qb, sq)
        pltpu.async_copy(kx_hbm.at[di], kb, sk)

    def step(ci, S, rm):
        si, di, qb, kb, lb, sq, sk, sl = S
        pltpu.make_async_copy(qx_hbm.at[si], qb, sq).wait()
        pltpu.make_async_copy(kx_hbm.at[di], kb, sk).wait()

        @pl.when(ci >= 2)
        def _():
            base_p = wid * EPW + (ci - 2) * C
            pltpu.make_async_copy(lb, lg_hbm.at[pl.ds(base_p, C)], sl).wait()

        def edge(e, rm):
            w = qb[e, pl.ds(0, L)] * kb[e, pl.ds(0, L)]
            for s in range(1, 8):
                w = w + qb[e, pl.ds(L * s, L)] * kb[e, pl.ds(L * s, L)]
            rot[pl.ds(0, L)] = w
            rot[pl.ds(L, L)] = w
            lv = (w + rot[pl.ds(8, L)]) * 0.25
            lb[e, :] = lv
            return jnp.maximum(rm, lv)

        rm = lax.fori_loop(0, C, edge, rm)
        base = wid * EPW + ci * C
        pltpu.async_copy(lb, lg_hbm.at[pl.ds(base, C)], sl)

        @pl.when(ci + 2 < NCHUNK)
        def _():
            start(ci + 2, S)

        return rm

    start(0, sets[0])
    start(1, sets[1])

    def pair(k, rm):
        rm = step(2 * k, sets[0], rm)
        rm = step(2 * k + 1, sets[1], rm)
        return rm

    rmax = lax.fori_loop(0, NCHUNK // 2, pair,
                         jnp.full((L,), -jnp.inf, jnp.float32))
    rmax = step(NCHUNK - 1, sets[0], rmax)
    b1 = wid * EPW + (NCHUNK - 2) * C
    pltpu.make_async_copy(lb1, lg_hbm.at[pl.ds(b1, C)], sl1).wait()
    b0 = wid * EPW + (NCHUNK - 1) * C
    pltpu.make_async_copy(lb0, lg_hbm.at[pl.ds(b0, C)], sl0).wait()
    rbuf[...] = rmax
    pltpu.sync_copy(rbuf, mx_hbm.at[wid])


@functools.partial(
    pl.kernel,
    mesh=_mesh,
    out_type=jax.ShapeDtypeStruct((NC * N, D), jnp.float32),
    scratch_types=[
        pltpu.VMEM((C,), jnp.int32),
        pltpu.VMEM((C,), jnp.int32),
        pltpu.VMEM((C,), jnp.int32),
        pltpu.VMEM((C, D), jnp.float32),
        pltpu.VMEM((C, D), jnp.float32),
        pltpu.VMEM((C, D), jnp.float32),
        pltpu.VMEM((C, L), jnp.float32),
        pltpu.VMEM((NW, L), jnp.float32),
        pltpu.VMEM((2 * L,), jnp.float32),
        pltpu.VMEM_SHARED((N, D), jnp.float32),
        pltpu.SemaphoreType.DMA,
        pltpu.SemaphoreType.DMA,
    ],
)
def _aggregate_pass(src_hbm, dst_hbm, vx_hbm, lg_hbm, mx_hbm, znd_hbm,
                    acc_hbm,
                    sidx, di0, di1, vb0, vb1, wbuf, lbuf,
                    mbuf, rot, acc_sh, sv0, sv1):
    cid = lax.axis_index("c")
    sid = lax.axis_index("s")
    wid = sid * NC + cid
    sets = ((di0, vb0, sv0), (di1, vb1, sv1))

    pltpu.sync_copy(mx_hbm, mbuf)
    gv = mbuf[0, :]
    for i in range(1, NW):
        gv = jnp.maximum(gv, mbuf[i, :])
    for off in (8, 4, 2, 1):
        rot[pl.ds(0, L)] = gv
        rot[pl.ds(L, L)] = gv
        gv = jnp.maximum(gv, rot[pl.ds(off, L)])

    row0 = sid * 624
    pltpu.sync_copy(znd_hbm.at[pl.ds(row0, 640)], acc_sh.at[pl.ds(row0, 640)])
    plsc.subcore_barrier()

    def start(ci, S):
        di, vb, sv = S
        base = wid * EPW + ci * C
        pltpu.sync_copy(dst_hbm.at[pl.ds(base, C)], di)
        pltpu.async_copy(vx_hbm.at[di], vb, sv)

    def step(ci, S):
        di, vb, sv = S
        base = wid * EPW + ci * C
        pltpu.make_async_copy(vx_hbm.at[di], vb, sv).wait()
        pltpu.sync_copy(lg_hbm.at[pl.ds(base, C)], lbuf)
        pltpu.sync_copy(src_hbm.at[pl.ds(base, C)], sidx)

        def edge(e, c2):
            ex = jnp.exp(lbuf[e, :] - gv)
            for s in range(8):
                wbuf[e, pl.ds(L * s, L)] = vb[e, pl.ds(L * s, L)] * ex
            return c2

        lax.fori_loop(0, C, edge, 0)
        pltpu.sync_copy(wbuf, acc_sh.at[sidx], add=True)

        @pl.when(ci + 2 < NCHUNK)
        def _():
            start(ci + 2, S)

    start(0, sets[0])
    start(1, sets[1])

    def pair(k, c2):
        step(2 * k, sets[0])
        step(2 * k + 1, sets[1])
        return c2

    lax.fori_loop(0, NCHUNK // 2, pair, 0)
    step(NCHUNK - 1, sets[0])
    plsc.subcore_barrier()

    @pl.when(sid == 0)
    def _():
        pltpu.sync_copy(acc_sh, acc_hbm.at[pl.ds(cid * N, N)])


@functools.partial(
    pl.kernel,
    mesh=_mesh,
    out_type=jax.ShapeDtypeStruct((NC * N, D), jnp.float32),
    scratch_types=[
        pltpu.VMEM((C,), jnp.int32),
        pltpu.VMEM((C, L), jnp.float32),
        pltpu.VMEM((C, L), jnp.float32),
        pltpu.VMEM((C, D), jnp.float32),
        pltpu.VMEM((NW, L), jnp.float32),
        pltpu.VMEM((2 * L,), jnp.float32),
        pltpu.VMEM_SHARED((N, D), jnp.float32),
        pltpu.SemaphoreType.DMA,
        pltpu.SemaphoreType.DMA,
    ],
)
def _denom_pass(src_hbm, lg_hbm, mx_hbm, znl_hbm, den_hbm,
                sidx, lb0, lb1, exbuf, mbuf, rot, den_sh, sg0, sg1):
    cid = lax.axis_index("c")
    sid = lax.axis_index("s")
    wid = sid * NC + cid
    sets = ((lb0, sg0), (lb1, sg1))

    pltpu.sync_copy(mx_hbm, mbuf)
    gv = mbuf[0, :]
    for i in range(1, NW):
        gv = jnp.maximum(gv, mbuf[i, :])
    for off in (8, 4, 2, 1):
        rot[pl.ds(0, L)] = gv
        rot[pl.ds(L, L)] = gv
        gv = jnp.maximum(gv, rot[pl.ds(off, L)])

    row0 = sid * 624
    pltpu.sync_copy(znl_hbm.at[pl.ds(row0, 640)], den_sh.at[pl.ds(row0, 640)])
    plsc.subcore_barrier()

    def start(ci, S):
        lb, sg = S
        base = wid * EPW + ci * C
        pltpu.async_copy(lg_hbm.at[pl.ds(base, C)], lb, sg)

    def step(ci, S):
        lb, sg = S
        base = wid * EPW + ci * C
        pltpu.make_async_copy(lg_hbm.at[pl.ds(base, C)], lb, sg).wait()
        pltpu.sync_copy(src_hbm.at[pl.ds(base, C)], sidx)

        def edge(e, c2):
            ex = jnp.exp(lb[e, :] - gv)
            for s in range(8):
                exbuf[e, pl.ds(L * s, L)] = ex
            return c2

        lax.fori_loop(0, C, edge, 0)
        pltpu.sync_copy(exbuf, den_sh.at[sidx], add=True)

        @pl.when(ci + 2 < NCHUNK)
        def _():
            start(ci + 2, S)

    start(0, sets[0])
    start(1, sets[1])

    def pair(k, c2):
        step(2 * k, sets[0])
        step(2 * k + 1, sets[1])
        return c2

    lax.fori_loop(0, NCHUNK // 2, pair, 0)
    step(NCHUNK - 1, sets[0])
    plsc.subcore_barrier()

    @pl.when(sid == 0)
    def _():
        pltpu.sync_copy(den_sh, den_hbm.at[pl.ds(cid * N, N)])


RB = 2000


def _ln_rows(v, w, b):
    mu = jnp.mean(v, axis=-1, keepdims=True)
    var = jnp.mean((v - mu) ** 2, axis=-1, keepdims=True)
    return (v - mu) / jnp.sqrt(var + EPS) * w + b


def _proj_body(x_ref, wq, wk, wv, bq, bk, bv, q_out, k_out, v_out):
    xb = x_ref[...]
    q_out[...] = jnp.dot(xb, wq[...], preferred_element_type=jnp.float32) + bq[...]
    k_out[...] = jnp.dot(xb, wk[...], preferred_element_type=jnp.float32) + bk[...]
    v_out[...] = jnp.dot(xb, wv[...], preferred_element_type=jnp.float32) + bv[...]


def _final_body(acc_ref, den_ref, s_ref, wout, bout, l1w, l1b, l2w, l2b, o_ref):
    acc = acc_ref[0] + acc_ref[1]
    den = den_ref[0] + den_ref[1]
    div = jnp.dot(den, s_ref[...], preferred_element_type=jnp.float32)
    attn = acc / (div + 1e-16)
    h1 = _ln_rows(attn, l1w[...], l1b[...])
    h2 = jnp.dot(h1, wout[...], preferred_element_type=jnp.float32) + bout[...]
    o_ref[...] = _ln_rows(h2, l2w[...], l2b[...])


def kernel(x, edge_index, WQ, bQ, WK, bK, WV, bV, Wout, bout,
           ln1_w, ln1_b, ln2_w, ln2_b):
    src = edge_index[0].astype(jnp.int32)
    dst = edge_index[1].astype(jnp.int32)

    def b2(v):
        return v.reshape(1, D).astype(jnp.float32)

    qx, kx, vx = pl.pallas_call(
        _proj_body,
        grid=(N // RB,),
        in_specs=[pl.BlockSpec((RB, D), lambda i: (i, 0))]
        + [pl.BlockSpec((D, D), lambda i: (0, 0))] * 3
        + [pl.BlockSpec((1, D), lambda i: (0, 0))] * 3,
        out_specs=[pl.BlockSpec((RB, D), lambda i: (i, 0))] * 3,
        out_shape=[jax.ShapeDtypeStruct((N, D), jnp.float32)] * 3,
    )(x, WQ[:, _PERM], WK[:, _PERM], WV[:, _PERM],
      b2(bQ[_PERM]), b2(bK[_PERM]), b2(bV[_PERM]))

    lg, mx = _logits_pass(src, dst, qx, kx)

    znd = jnp.zeros((N, D), jnp.float32)
    accp = _aggregate_pass(src, dst, vx, lg, mx, znd)
    denp = _denom_pass(src, lg, mx, znd)
    acc3 = accp.reshape(NC, N, D)
    den3 = denp.reshape(NC, N, D)

    out = pl.pallas_call(
        _final_body,
        grid=(N // RB,),
        in_specs=[
            pl.BlockSpec((NC, RB, D), lambda i: (0, i, 0)),
            pl.BlockSpec((NC, RB, D), lambda i: (0, i, 0)),
            pl.BlockSpec((D, D), lambda i: (0, 0)),
            pl.BlockSpec((D, D), lambda i: (0, 0)),
            pl.BlockSpec((1, D), lambda i: (0, 0)),
            pl.BlockSpec((1, D), lambda i: (0, 0)),
            pl.BlockSpec((1, D), lambda i: (0, 0)),
            pl.BlockSpec((1, D), lambda i: (0, 0)),
            pl.BlockSpec((1, D), lambda i: (0, 0)),
        ],
        out_specs=pl.BlockSpec((RB, D), lambda i: (i, 0)),
        out_shape=jax.ShapeDtypeStruct((N, D), jnp.float32),
    )(acc3, den3, _S128, Wout[_PERM, :], b2(bout),
      b2(ln1_w[_PERM]), b2(ln1_b[_PERM]), b2(ln2_w), b2(ln2_b))
    return out
